# baseline (speedup 1.0000x reference)
import functools

import jax
import jax.numpy as jnp
from jax import lax
from jax.experimental import pallas as pl
from jax.experimental.pallas import tpu as pltpu
from jax.experimental.pallas import tpu_sc as plsc

_INFO = plsc.get_sparse_core_info()
_NC, _NS, _L = _INFO.num_cores, _INFO.num_subcores, _INFO.num_lanes
_NW = _NC * _NS

_CHUNK = 8192
_UNROLL = 8
_NIB = 4
_NOB = 2
_MAGIC = 1.5 * 2.0 ** 23


@functools.lru_cache(maxsize=None)
def _make_quantize(n: int, k: int):
    per_w = n // _NW
    assert n % (_NW * _L) == 0 and per_w % _CHUNK == 0
    assert k >= _L
    nch = per_w // _CHUNK
    n_vec = _CHUNK // _L
    assert n_vec % _UNROLL == 0
    mesh = plsc.VectorSubcoreMesh(core_axis_name="c", subcore_axis_name="s")

    @functools.partial(
        pl.kernel,
        mesh=mesh,
        out_type=jax.ShapeDtypeStruct((n,), jnp.float32),
        scratch_types=[
            pltpu.VMEM((_NIB, _CHUNK), jnp.float32),
            pltpu.VMEM((_NOB, _CHUNK), jnp.float32),
            pltpu.VMEM((k,), jnp.float32),
        ] + [pltpu.SemaphoreType.DMA] * (_NIB + _NOB),
    )
    def _quantize(x_hbm, cb_hbm, out_hbm, ibuf, obuf, cb_v, *sems):
        isems = sems[:_NIB]
        osems = sems[_NIB:]
        wid = lax.axis_index("s") * _NC + lax.axis_index("c")
        base = wid * per_w
        pltpu.sync_copy(cb_hbm, cb_v)

        def bcast_lane(vec, lane):
            idx = jnp.full((_L, 1), lane, jnp.int32)
            dn = lax.GatherDimensionNumbers(
                offset_dims=(), collapsed_slice_dims=(0,),
                start_index_map=(0,))
            return lax.gather(vec, idx, dn, (1,),
                              mode=lax.GatherScatterMode.PROMISE_IN_BOUNDS)

        c0 = bcast_lane(cb_v[pl.ds(0, _L)], 0)
        c_last = bcast_lane(cb_v[pl.ds(k - _L, _L)], _L - 1)
        step = (c_last - c0) * jnp.float32(1.0 / (k - 1))
        inv = jnp.float32(1.0) / step
        bias = -c0 * inv
        kmax = jnp.full((_L,), float(k - 1), jnp.float32)
        zero = jnp.zeros((_L,), jnp.float32)
        magic = jnp.full((_L,), _MAGIC, jnp.float32)

        def in_dma(j):
            return pltpu.async_copy(
                x_hbm.at[pl.ds(base + j * _CHUNK, _CHUNK)],
                ibuf.at[j % _NIB], isems[j % _NIB])

        def out_dma(j):
            return pltpu.async_copy(
                obuf.at[j % _NOB],
                out_hbm.at[pl.ds(base + j * _CHUNK, _CHUNK)], osems[j % _NOB])

        def compute(src, dst):
            def body(i, carry):
                for u in range(_UNROLL):
                    off = (i * _UNROLL + u) * _L
                    x = src[pl.ds(off, _L)]
                    t = x * inv + bias
                    t = jnp.minimum(jnp.maximum(t, zero), kmax)
                    r = (t + magic) - magic
                    dst[pl.ds(off, _L)] = r * step + c0
                return carry

            lax.fori_loop(0, n_vec // _UNROLL, body, 0)

        hin = [None] * _NIB
        hout = [None] * _NOB
        for j in range(min(_NIB - 1, nch)):
            hin[j % _NIB] = in_dma(j)
        for j in range(nch):
            ib, ob = j % _NIB, j % _NOB
            hin[ib].wait()
            nj = j + _NIB - 1
            if nj < nch:
                hin[nj % _NIB] = in_dma(nj)
            if j >= _NOB:
                hout[ob].wait()
            compute(ibuf.at[ib], obuf.at[ob])
            hout[ob] = out_dma(j)
        for h in hout:
            if h is not None:
                h.wait()

    return _quantize


def kernel(input, codebook):
    n = input.size
    k = codebook.shape[0]
    out = _make_quantize(n, k)(input.reshape(n), codebook)
    return out.reshape(input.shape)


# speedup vs baseline: 1.7235x; 1.7235x over previous
import functools

import jax
import jax.numpy as jnp
from jax import lax
from jax.experimental import pallas as pl
from jax.experimental.pallas import tpu as pltpu
from jax.experimental.pallas import tpu_sc as plsc

_INFO = plsc.get_sparse_core_info()
_NC, _NS, _L = _INFO.num_cores, _INFO.num_subcores, _INFO.num_lanes
_NW = _NC * _NS

_CHUNK = 8192
_UNROLL = 8


@functools.lru_cache(maxsize=None)
def _make_quantize(n: int, k: int):
    per_w = n // _NW
    assert n % (_NW * _L) == 0 and per_w % _CHUNK == 0
    nch = per_w // _CHUNK
    n_vec = _CHUNK // _L
    assert n_vec % _UNROLL == 0
    mesh = plsc.VectorSubcoreMesh(core_axis_name="c", subcore_axis_name="s")

    @functools.partial(
        pl.kernel,
        mesh=mesh,
        out_type=jax.ShapeDtypeStruct((n,), jnp.float32),
        scratch_types=[
            pltpu.VMEM((2, _CHUNK), jnp.float32),
            pltpu.VMEM((2, _CHUNK), jnp.float32),
            pltpu.VMEM((4 * _L,), jnp.float32),
            pltpu.SemaphoreType.DMA,
            pltpu.SemaphoreType.DMA,
            pltpu.SemaphoreType.DMA,
            pltpu.SemaphoreType.DMA,
        ],
    )
    def _quantize(x_hbm, consts_hbm, out_hbm, ibuf, obuf, consts_v,
                  isem0, isem1, osem0, osem1):
        isems = (isem0, isem1)
        osems = (osem0, osem1)
        wid = lax.axis_index("s") * _NC + lax.axis_index("c")
        base = wid * per_w
        pltpu.sync_copy(consts_hbm, consts_v)
        bias = consts_v[pl.ds(0, _L)]
        c0 = consts_v[pl.ds(_L, _L)]
        step = consts_v[pl.ds(2 * _L, _L)]
        inv = consts_v[pl.ds(3 * _L, _L)]
        ubound = jnp.full((_L,), (k - 1) + 0.4999, jnp.float32)
        zero = jnp.zeros((_L,), jnp.float32)

        def in_dma(j):
            return pltpu.async_copy(
                x_hbm.at[pl.ds(base + j * _CHUNK, _CHUNK)],
                ibuf.at[j % 2], isems[j % 2])

        def out_dma(j):
            return pltpu.async_copy(
                obuf.at[j % 2],
                out_hbm.at[pl.ds(base + j * _CHUNK, _CHUNK)], osems[j % 2])

        def compute(b):
            src = ibuf.at[b]
            dst = obuf.at[b]

            def body(i, carry):
                for u in range(_UNROLL):
                    off = (i * _UNROLL + u) * _L
                    x = src[pl.ds(off, _L)]
                    t = x * inv + bias
                    t = jnp.minimum(jnp.maximum(t, zero), ubound)
                    idx_f = t.astype(jnp.int32).astype(jnp.float32)
                    dst[pl.ds(off, _L)] = c0 + idx_f * step
                return carry

            lax.fori_loop(0, n_vec // _UNROLL, body, 0)

        hin = [None, None]
        hout = [None, None]
        hin[0] = in_dma(0)
        if nch > 1:
            hin[1] = in_dma(1)
        for j in range(nch):
            b = j % 2
            if j >= 2:
                hout[b].wait()
            hin[b].wait()
            compute(b)
            hout[b] = out_dma(j)
            if j + 2 < nch:
                hin[b] = in_dma(j + 2)
        if nch > 1:
            hout[(nch - 2) % 2].wait()
        hout[(nch - 1) % 2].wait()

    return _quantize


def kernel(input, codebook):
    n = input.size
    k = codebook.shape[0]
    c0 = codebook[0]
    span = codebook[k - 1] - codebook[0]
    step = span / (k - 1)
    inv_step = (k - 1) / span
    consts = jnp.concatenate([
        jnp.broadcast_to(0.5 - c0 * inv_step, (_L,)),
        jnp.broadcast_to(c0, (_L,)),
        jnp.broadcast_to(step, (_L,)),
        jnp.broadcast_to(inv_step, (_L,)),
    ]).astype(jnp.float32)
    out = _make_quantize(n, k)(input.reshape(n), consts)
    return out.reshape(input.shape)


# speedup vs baseline: 1.7246x; 1.0006x over previous
import functools

import jax
import jax.numpy as jnp
from jax import lax
from jax.experimental import pallas as pl
from jax.experimental.pallas import tpu as pltpu
from jax.experimental.pallas import tpu_sc as plsc

_INFO = plsc.get_sparse_core_info()
_NC, _NS, _L = _INFO.num_cores, _INFO.num_subcores, _INFO.num_lanes
_NW = _NC * _NS

_CHUNK = 8192
_UNROLL = 8


@functools.lru_cache(maxsize=None)
def _make_quantize(n: int, k: int):
    per_w = n // _NW
    assert n % (_NW * _L) == 0 and per_w % _CHUNK == 0
    nch = per_w // _CHUNK
    n_vec = _CHUNK // _L
    assert n_vec % _UNROLL == 0
    mesh = plsc.VectorSubcoreMesh(core_axis_name="c", subcore_axis_name="s")

    @functools.partial(
        pl.kernel,
        mesh=mesh,
        out_type=jax.ShapeDtypeStruct((n,), jnp.float32),
        scratch_types=[
            pltpu.VMEM((2, _CHUNK), jnp.float32),
            pltpu.VMEM((2, _CHUNK), jnp.float32),
            pltpu.VMEM((4 * _L,), jnp.float32),
            pltpu.VMEM((k,), jnp.float32),
            pltpu.SemaphoreType.DMA,
            pltpu.SemaphoreType.DMA,
            pltpu.SemaphoreType.DMA,
            pltpu.SemaphoreType.DMA,
        ],
    )
    def _quantize(x_hbm, cb_hbm, out_hbm, ibuf, obuf, consts_v, cb_v,
                  isem0, isem1, osem0, osem1):
        isems = (isem0, isem1)
        osems = (osem0, osem1)
        wid = lax.axis_index("s") * _NC + lax.axis_index("c")
        base = wid * per_w
        pltpu.sync_copy(cb_hbm, cb_v)

        def bcast_lane(vec, lane):
            idx = jnp.full((_L, 1), lane, jnp.int32)
            dn = lax.GatherDimensionNumbers(
                offset_dims=(), collapsed_slice_dims=(0,),
                start_index_map=(0,))
            return lax.gather(vec, idx, dn, (1,),
                              mode=lax.GatherScatterMode.PROMISE_IN_BOUNDS)

        c0_g = bcast_lane(cb_v[pl.ds(0, _L)], 0)
        c_last = bcast_lane(cb_v[pl.ds(k - _L, _L)], _L - 1)
        step_g = (c_last - c0_g) * jnp.float32(1.0 / (k - 1))
        inv_g = jnp.float32(1.0) / step_g
        consts_v[pl.ds(0, _L)] = jnp.full((_L,), 0.5, jnp.float32) - c0_g * inv_g
        consts_v[pl.ds(_L, _L)] = c0_g
        consts_v[pl.ds(2 * _L, _L)] = step_g
        consts_v[pl.ds(3 * _L, _L)] = inv_g
        bias = consts_v[pl.ds(0, _L)]
        c0 = consts_v[pl.ds(_L, _L)]
        step = consts_v[pl.ds(2 * _L, _L)]
        inv = consts_v[pl.ds(3 * _L, _L)]
        ubound = jnp.full((_L,), (k - 1) + 0.4999, jnp.float32)
        zero = jnp.zeros((_L,), jnp.float32)

        def in_dma(j):
            return pltpu.async_copy(
                x_hbm.at[pl.ds(base + j * _CHUNK, _CHUNK)],
                ibuf.at[j % 2], isems[j % 2])

        def out_dma(j):
            return pltpu.async_copy(
                obuf.at[j % 2],
                out_hbm.at[pl.ds(base + j * _CHUNK, _CHUNK)], osems[j % 2])

        def compute(b):
            src = ibuf.at[b]
            dst = obuf.at[b]

            def body(i, carry):
                for u in range(_UNROLL):
                    off = (i * _UNROLL + u) * _L
                    x = src[pl.ds(off, _L)]
                    t = x * inv + bias
                    t = jnp.minimum(jnp.maximum(t, zero), ubound)
                    idx_f = t.astype(jnp.int32).astype(jnp.float32)
                    dst[pl.ds(off, _L)] = c0 + idx_f * step
                return carry

            lax.fori_loop(0, n_vec // _UNROLL, body, 0)

        hin = [None, None]
        hout = [None, None]
        hin[0] = in_dma(0)
        if nch > 1:
            hin[1] = in_dma(1)
        for j in range(nch):
            b = j % 2
            if j >= 2:
                hout[b].wait()
            hin[b].wait()
            compute(b)
            hout[b] = out_dma(j)
            if j + 2 < nch:
                hin[b] = in_dma(j + 2)
        if nch > 1:
            hout[(nch - 2) % 2].wait()
        hout[(nch - 1) % 2].wait()

    return _quantize


def kernel(input, codebook):
    n = input.size
    k = codebook.shape[0]
    out = _make_quantize(n, k)(input.reshape(n), codebook)
    return out.reshape(input.shape)


# speedup vs baseline: 2.3163x; 1.3431x over previous
import functools

import jax
import jax.numpy as jnp
from jax import lax
from jax.experimental import pallas as pl
from jax.experimental.pallas import tpu as pltpu
from jax.experimental.pallas import tpu_sc as plsc

_INFO = plsc.get_sparse_core_info()
_NC, _NS, _L = _INFO.num_cores, _INFO.num_subcores, _INFO.num_lanes
_NW = _NC * _NS

_RCH = 8


@functools.lru_cache(maxsize=None)
def _make_quantize(nrow: int, ncol: int, k: int):
    rows_w = nrow // _NW
    assert nrow % _NW == 0 and rows_w % _RCH == 0 and ncol % _L == 0
    assert k >= _L
    nch = rows_w // _RCH
    n_vec = ncol // _L
    mesh = plsc.VectorSubcoreMesh(core_axis_name="c", subcore_axis_name="s")

    @functools.partial(
        pl.kernel,
        mesh=mesh,
        out_type=jax.ShapeDtypeStruct((nrow, ncol), jnp.float32),
        scratch_types=[
            pltpu.VMEM((2, _RCH, ncol), jnp.float32),
            pltpu.VMEM((2, _RCH, ncol), jnp.float32),
            pltpu.VMEM((4 * _L,), jnp.float32),
            pltpu.VMEM((k,), jnp.float32),
            pltpu.SemaphoreType.DMA,
            pltpu.SemaphoreType.DMA,
            pltpu.SemaphoreType.DMA,
            pltpu.SemaphoreType.DMA,
        ],
    )
    def _quantize(x_hbm, cb_hbm, out_hbm, ibuf, obuf, consts_v, cb_v,
                  isem0, isem1, osem0, osem1):
        isems = (isem0, isem1)
        osems = (osem0, osem1)
        wid = lax.axis_index("s") * _NC + lax.axis_index("c")
        row0 = wid * rows_w
        pltpu.sync_copy(cb_hbm, cb_v)

        def bcast_lane(vec, lane):
            idx = jnp.full((_L, 1), lane, jnp.int32)
            dn = lax.GatherDimensionNumbers(
                offset_dims=(), collapsed_slice_dims=(0,),
                start_index_map=(0,))
            return lax.gather(vec, idx, dn, (1,),
                              mode=lax.GatherScatterMode.PROMISE_IN_BOUNDS)

        c0_g = bcast_lane(cb_v[pl.ds(0, _L)], 0)
        c_last = bcast_lane(cb_v[pl.ds(k - _L, _L)], _L - 1)
        step_g = (c_last - c0_g) * jnp.float32(1.0 / (k - 1))
        inv_g = jnp.float32(1.0) / step_g
        consts_v[pl.ds(0, _L)] = jnp.full((_L,), 0.5, jnp.float32) - c0_g * inv_g
        consts_v[pl.ds(_L, _L)] = c0_g
        consts_v[pl.ds(2 * _L, _L)] = step_g
        consts_v[pl.ds(3 * _L, _L)] = inv_g
        bias = consts_v[pl.ds(0, _L)]
        c0 = consts_v[pl.ds(_L, _L)]
        step = consts_v[pl.ds(2 * _L, _L)]
        inv = consts_v[pl.ds(3 * _L, _L)]
        ubound = jnp.full((_L,), (k - 1) + 0.4999, jnp.float32)
        zero = jnp.zeros((_L,), jnp.float32)

        def in_dma(j):
            return pltpu.async_copy(
                x_hbm.at[pl.ds(row0 + j * _RCH, _RCH)],
                ibuf.at[j % 2], isems[j % 2])

        def out_dma(j):
            return pltpu.async_copy(
                obuf.at[j % 2],
                out_hbm.at[pl.ds(row0 + j * _RCH, _RCH)], osems[j % 2])

        def compute(b):
            def body(i, carry):
                for r in range(_RCH):
                    x = ibuf[b, r, pl.ds(i * _L, _L)]
                    t = x * inv + bias
                    t = jnp.minimum(jnp.maximum(t, zero), ubound)
                    idx_f = t.astype(jnp.int32).astype(jnp.float32)
                    obuf[b, r, pl.ds(i * _L, _L)] = c0 + idx_f * step
                return carry

            lax.fori_loop(0, n_vec, body, 0)

        hin = [None, None]
        hout = [None, None]
        hin[0] = in_dma(0)
        if nch > 1:
            hin[1] = in_dma(1)
        for j in range(nch):
            b = j % 2
            if j >= 2:
                hout[b].wait()
            hin[b].wait()
            compute(b)
            hout[b] = out_dma(j)
            if j + 2 < nch:
                hin[b] = in_dma(j + 2)
        if nch > 1:
            hout[(nch - 2) % 2].wait()
        hout[(nch - 1) % 2].wait()

    return _quantize


def kernel(input, codebook):
    nrow, ncol = input.shape
    k = codebook.shape[0]
    return _make_quantize(nrow, ncol, k)(input, codebook)


# speedup vs baseline: 2.4651x; 1.0642x over previous
import functools

import jax
import jax.numpy as jnp
from jax import lax
from jax.experimental import pallas as pl
from jax.experimental.pallas import tpu as pltpu
from jax.experimental.pallas import tpu_sc as plsc

_INFO = plsc.get_sparse_core_info()
_NC, _NS, _L = _INFO.num_cores, _INFO.num_subcores, _INFO.num_lanes
_NW = _NC * _NS

_RCH = 16
_MAGIC = 1.5 * 2.0 ** 23


@functools.lru_cache(maxsize=None)
def _make_quantize(nrow: int, ncol: int, k: int):
    rows_w = nrow // _NW
    assert nrow % _NW == 0 and rows_w % _RCH == 0 and ncol % _L == 0
    assert k >= _L
    nch = rows_w // _RCH
    n_vec = ncol // _L
    mesh = plsc.VectorSubcoreMesh(core_axis_name="c", subcore_axis_name="s")

    @functools.partial(
        pl.kernel,
        mesh=mesh,
        out_type=jax.ShapeDtypeStruct((nrow, ncol), jnp.float32),
        scratch_types=[
            pltpu.VMEM((2, _RCH, ncol), jnp.float32),
            pltpu.VMEM((2, _RCH, ncol), jnp.float32),
            pltpu.VMEM((4 * _L,), jnp.float32),
            pltpu.VMEM((k,), jnp.float32),
            pltpu.SemaphoreType.DMA,
            pltpu.SemaphoreType.DMA,
            pltpu.SemaphoreType.DMA,
            pltpu.SemaphoreType.DMA,
        ],
    )
    def _quantize(x_hbm, cb_hbm, out_hbm, ibuf, obuf, consts_v, cb_v,
                  isem0, isem1, osem0, osem1):
        isems = (isem0, isem1)
        osems = (osem0, osem1)
        wid = lax.axis_index("s") * _NC + lax.axis_index("c")
        row0 = wid * rows_w
        pltpu.sync_copy(cb_hbm, cb_v)

        def bcast_lane(vec, lane):
            idx = jnp.full((_L, 1), lane, jnp.int32)
            dn = lax.GatherDimensionNumbers(
                offset_dims=(), collapsed_slice_dims=(0,),
                start_index_map=(0,))
            return lax.gather(vec, idx, dn, (1,),
                              mode=lax.GatherScatterMode.PROMISE_IN_BOUNDS)

        c0_g = bcast_lane(cb_v[pl.ds(0, _L)], 0)
        c_last = bcast_lane(cb_v[pl.ds(k - _L, _L)], _L - 1)
        step_g = (c_last - c0_g) * jnp.float32(1.0 / (k - 1))
        inv_g = jnp.float32(1.0) / step_g
        consts_v[pl.ds(0, _L)] = -c0_g * inv_g
        consts_v[pl.ds(_L, _L)] = c0_g
        consts_v[pl.ds(2 * _L, _L)] = step_g
        consts_v[pl.ds(3 * _L, _L)] = inv_g
        bias = consts_v[pl.ds(0, _L)]
        c0 = consts_v[pl.ds(_L, _L)]
        step = consts_v[pl.ds(2 * _L, _L)]
        inv = consts_v[pl.ds(3 * _L, _L)]
        ubound = jnp.full((_L,), float(k - 1), jnp.float32)
        zero = jnp.zeros((_L,), jnp.float32)
        magic = jnp.full((_L,), _MAGIC, jnp.float32)

        def in_dma(j):
            return pltpu.async_copy(
                x_hbm.at[pl.ds(row0 + j * _RCH, _RCH)],
                ibuf.at[j % 2], isems[j % 2])

        def out_dma(j):
            return pltpu.async_copy(
                obuf.at[j % 2],
                out_hbm.at[pl.ds(row0 + j * _RCH, _RCH)], osems[j % 2])

        def compute(b):
            def body(i, carry):
                for r in range(_RCH):
                    x = ibuf[b, r, pl.ds(i * _L, _L)]
                    t = x * inv + bias
                    t = jnp.minimum(jnp.maximum(t, zero), ubound)
                    idx_f = (t + magic) - magic
                    obuf[b, r, pl.ds(i * _L, _L)] = c0 + idx_f * step
                return carry

            lax.fori_loop(0, n_vec, body, 0)

        hin = [None, None]
        hout = [None, None]
        hin[0] = in_dma(0)
        if nch > 1:
            hin[1] = in_dma(1)
        for j in range(nch):
            b = j % 2
            if j >= 2:
                hout[b].wait()
            hin[b].wait()
            compute(b)
            hout[b] = out_dma(j)
            if j + 2 < nch:
                hin[b] = in_dma(j + 2)
        if nch > 1:
            hout[(nch - 2) % 2].wait()
        hout[(nch - 1) % 2].wait()

    return _quantize


def kernel(input, codebook):
    nrow, ncol = input.shape
    k = codebook.shape[0]
    return _make_quantize(nrow, ncol, k)(input, codebook)


# speedup vs baseline: 2.7796x; 1.1276x over previous
import functools

import jax
import jax.numpy as jnp
from jax import lax
from jax.experimental import pallas as pl
from jax.experimental.pallas import tpu as pltpu
from jax.experimental.pallas import tpu_sc as plsc

_INFO = plsc.get_sparse_core_info()
_NC, _NS, _L = _INFO.num_cores, _INFO.num_subcores, _INFO.num_lanes
_NW = _NC * _NS

_NB = 4
_MAGIC = 1.5 * 2.0 ** 23


@functools.lru_cache(maxsize=None)
def _make_quantize(nrow: int, ncol: int, k: int):
    rows_w = nrow // _NW
    rch = rows_w // _NB
    assert nrow % _NW == 0 and rows_w % _NB == 0 and ncol % _L == 0
    assert k >= _L
    n_vec = ncol // _L
    mesh = plsc.VectorSubcoreMesh(core_axis_name="c", subcore_axis_name="s")

    @functools.partial(
        pl.kernel,
        mesh=mesh,
        out_type=jax.ShapeDtypeStruct((nrow, ncol), jnp.float32),
        scratch_types=[pltpu.VMEM((rch, ncol), jnp.float32)] * _NB + [
            pltpu.VMEM((4 * _L,), jnp.float32),
            pltpu.VMEM((k,), jnp.float32),
        ] + [pltpu.SemaphoreType.DMA] * (2 * _NB),
    )
    def _quantize(x_hbm, cb_hbm, out_hbm, *refs):
        bufs = refs[:_NB]
        consts_v = refs[_NB]
        cb_v = refs[_NB + 1]
        isems = refs[_NB + 2:2 * _NB + 2]
        osems = refs[2 * _NB + 2:]
        wid = lax.axis_index("s") * _NC + lax.axis_index("c")
        row0 = wid * rows_w

        hin = [
            pltpu.async_copy(
                x_hbm.at[pl.ds(row0 + j * rch, rch)], bufs[j], isems[j])
            for j in range(_NB)
        ]

        pltpu.sync_copy(cb_hbm, cb_v)

        def bcast_lane(vec, lane):
            idx = jnp.full((_L, 1), lane, jnp.int32)
            dn = lax.GatherDimensionNumbers(
                offset_dims=(), collapsed_slice_dims=(0,),
                start_index_map=(0,))
            return lax.gather(vec, idx, dn, (1,),
                              mode=lax.GatherScatterMode.PROMISE_IN_BOUNDS)

        c0_g = bcast_lane(cb_v[pl.ds(0, _L)], 0)
        c_last = bcast_lane(cb_v[pl.ds(k - _L, _L)], _L - 1)
        step_g = (c_last - c0_g) * jnp.float32(1.0 / (k - 1))
        inv_g = jnp.float32(1.0) / step_g
        consts_v[pl.ds(0, _L)] = -c0_g * inv_g
        consts_v[pl.ds(_L, _L)] = c0_g
        consts_v[pl.ds(2 * _L, _L)] = step_g
        consts_v[pl.ds(3 * _L, _L)] = inv_g
        bias = consts_v[pl.ds(0, _L)]
        c0 = consts_v[pl.ds(_L, _L)]
        step = consts_v[pl.ds(2 * _L, _L)]
        inv = consts_v[pl.ds(3 * _L, _L)]
        ubound = jnp.full((_L,), float(k - 1), jnp.float32)
        zero = jnp.zeros((_L,), jnp.float32)
        magic = jnp.full((_L,), _MAGIC, jnp.float32)

        hout = []
        for j in range(_NB):
            hin[j].wait()
            buf = bufs[j]

            def body(i, carry, buf=buf):
                for r in range(rch):
                    x = buf[r, pl.ds(i * _L, _L)]
                    t = x * inv + bias
                    t = jnp.minimum(jnp.maximum(t, zero), ubound)
                    idx_f = (t + magic) - magic
                    buf[r, pl.ds(i * _L, _L)] = c0 + idx_f * step
                return carry

            lax.fori_loop(0, n_vec, body, 0)
            hout.append(pltpu.async_copy(
                buf, out_hbm.at[pl.ds(row0 + j * rch, rch)], osems[j]))
        for h in hout:
            h.wait()

    return _quantize


def kernel(input, codebook):
    nrow, ncol = input.shape
    k = codebook.shape[0]
    return _make_quantize(nrow, ncol, k)(input, codebook)


# speedup vs baseline: 3.0758x; 1.1066x over previous
import functools

import jax
import jax.numpy as jnp
from jax import lax
from jax.experimental import pallas as pl
from jax.experimental.pallas import tpu as pltpu
from jax.experimental.pallas import tpu_sc as plsc

_INFO = plsc.get_sparse_core_info()
_NC, _NS, _L = _INFO.num_cores, _INFO.num_subcores, _INFO.num_lanes
_NW = _NC * _NS

_NB = 4
_MAGIC = 1.5 * 2.0 ** 23


@functools.lru_cache(maxsize=None)
def _make_quantize(nrow: int, ncol: int, k: int):
    rows_w = nrow // _NW
    rch = rows_w // _NB
    assert nrow % _NW == 0 and rows_w % _NB == 0 and ncol % _L == 0
    assert k >= _L
    n_vec = ncol // _L
    mesh = plsc.VectorSubcoreMesh(core_axis_name="c", subcore_axis_name="s")

    @functools.partial(
        pl.kernel,
        mesh=mesh,
        out_type=jax.ShapeDtypeStruct((nrow, ncol), jnp.float32),
        scratch_types=[pltpu.VMEM((rch, ncol), jnp.float32)] * _NB + [
            pltpu.VMEM((4 * _L,), jnp.float32),
            pltpu.VMEM((k,), jnp.float32),
        ] + [pltpu.SemaphoreType.DMA] * (2 * _NB),
    )
    def _quantize(x_hbm, cb_hbm, out_hbm, *refs):
        bufs = refs[:_NB]
        consts_v = refs[_NB]
        cb_v = refs[_NB + 1]
        isems = refs[_NB + 2:2 * _NB + 2]
        osems = refs[2 * _NB + 2:]
        wid = lax.axis_index("s") * _NC + lax.axis_index("c")
        row0 = wid * rows_w

        hin = [
            pltpu.async_copy(
                x_hbm.at[pl.ds(row0 + j * rch, rch)], bufs[j], isems[j])
            for j in range(_NB)
        ]

        pltpu.sync_copy(cb_hbm, cb_v)

        def bcast_lane(vec, lane):
            idx = jnp.full((_L, 1), lane, jnp.int32)
            dn = lax.GatherDimensionNumbers(
                offset_dims=(), collapsed_slice_dims=(0,),
                start_index_map=(0,))
            return lax.gather(vec, idx, dn, (1,),
                              mode=lax.GatherScatterMode.PROMISE_IN_BOUNDS)

        c0_g = bcast_lane(cb_v[pl.ds(0, _L)], 0)
        c_last = bcast_lane(cb_v[pl.ds(k - _L, _L)], _L - 1)
        step_g = (c_last - c0_g) * jnp.float32(1.0 / (k - 1))
        inv_g = jnp.float32(1.0) / step_g
        consts_v[pl.ds(0, _L)] = -c0_g * inv_g
        consts_v[pl.ds(_L, _L)] = c0_g
        consts_v[pl.ds(2 * _L, _L)] = step_g
        consts_v[pl.ds(3 * _L, _L)] = inv_g
        bias = consts_v[pl.ds(0, _L)]
        c0 = consts_v[pl.ds(_L, _L)]
        step = consts_v[pl.ds(2 * _L, _L)]
        inv = consts_v[pl.ds(3 * _L, _L)]
        ubound = jnp.full((_L,), float(k - 1), jnp.float32)
        zero = jnp.zeros((_L,), jnp.float32)
        magic = jnp.full((_L,), _MAGIC, jnp.float32)

        hout = []
        for j in range(_NB):
            hin[j].wait()
            buf = bufs[j]

            @plsc.parallel_loop(0, n_vec)
            def _loop(i, buf=buf):
                for r in range(rch):
                    x = buf[r, pl.ds(i * _L, _L)]
                    t = x * inv + bias
                    t = jnp.minimum(jnp.maximum(t, zero), ubound)
                    idx_f = (t + magic) - magic
                    buf[r, pl.ds(i * _L, _L)] = c0 + idx_f * step
            hout.append(pltpu.async_copy(
                buf, out_hbm.at[pl.ds(row0 + j * rch, rch)], osems[j]))
        for h in hout:
            h.wait()

    return _quantize


def kernel(input, codebook):
    nrow, ncol = input.shape
    k = codebook.shape[0]
    return _make_quantize(nrow, ncol, k)(input, codebook)


# speedup vs baseline: 3.1214x; 1.0148x over previous
import functools

import jax
import jax.numpy as jnp
from jax import lax
from jax.experimental import pallas as pl
from jax.experimental.pallas import tpu as pltpu
from jax.experimental.pallas import tpu_sc as plsc

_INFO = plsc.get_sparse_core_info()
_NC, _NS, _L = _INFO.num_cores, _INFO.num_subcores, _INFO.num_lanes
_NW = _NC * _NS

_NB = 8
_MAGIC = 1.5 * 2.0 ** 23


@functools.lru_cache(maxsize=None)
def _make_quantize(nrow: int, ncol: int, k: int):
    rows_w = nrow // _NW
    rch = rows_w // _NB
    assert nrow % _NW == 0 and rows_w % _NB == 0 and ncol % _L == 0
    assert k >= _L
    n_vec = ncol // _L
    mesh = plsc.VectorSubcoreMesh(core_axis_name="c", subcore_axis_name="s")

    @functools.partial(
        pl.kernel,
        mesh=mesh,
        out_type=jax.ShapeDtypeStruct((nrow, ncol), jnp.float32),
        scratch_types=[pltpu.VMEM((rch, ncol), jnp.float32)] * _NB + [
            pltpu.VMEM((4 * _L,), jnp.float32),
            pltpu.VMEM((k,), jnp.float32),
        ] + [pltpu.SemaphoreType.DMA] * (2 * _NB),
    )
    def _quantize(x_hbm, cb_hbm, out_hbm, *refs):
        bufs = refs[:_NB]
        consts_v = refs[_NB]
        cb_v = refs[_NB + 1]
        isems = refs[_NB + 2:2 * _NB + 2]
        osems = refs[2 * _NB + 2:]
        wid = lax.axis_index("s") * _NC + lax.axis_index("c")
        row0 = wid * rows_w

        hin = [
            pltpu.async_copy(
                x_hbm.at[pl.ds(row0 + j * rch, rch)], bufs[j], isems[j])
            for j in range(_NB)
        ]

        pltpu.sync_copy(cb_hbm, cb_v)

        def bcast_lane(vec, lane):
            idx = jnp.full((_L, 1), lane, jnp.int32)
            dn = lax.GatherDimensionNumbers(
                offset_dims=(), collapsed_slice_dims=(0,),
                start_index_map=(0,))
            return lax.gather(vec, idx, dn, (1,),
                              mode=lax.GatherScatterMode.PROMISE_IN_BOUNDS)

        c0_g = bcast_lane(cb_v[pl.ds(0, _L)], 0)
        c_last = bcast_lane(cb_v[pl.ds(k - _L, _L)], _L - 1)
        step_g = (c_last - c0_g) * jnp.float32(1.0 / (k - 1))
        inv_g = jnp.float32(1.0) / step_g
        consts_v[pl.ds(0, _L)] = -c0_g * inv_g
        consts_v[pl.ds(_L, _L)] = c0_g
        consts_v[pl.ds(2 * _L, _L)] = step_g
        consts_v[pl.ds(3 * _L, _L)] = inv_g
        bias = consts_v[pl.ds(0, _L)]
        c0 = consts_v[pl.ds(_L, _L)]
        step = consts_v[pl.ds(2 * _L, _L)]
        inv = consts_v[pl.ds(3 * _L, _L)]
        ubound = jnp.full((_L,), float(k - 1), jnp.float32)
        zero = jnp.zeros((_L,), jnp.float32)
        magic = jnp.full((_L,), _MAGIC, jnp.float32)

        hout = []
        for j in range(_NB):
            hin[j].wait()
            buf = bufs[j]

            @plsc.parallel_loop(0, n_vec)
            def _loop(i, buf=buf):
                for r in range(rch):
                    x = buf[r, pl.ds(i * _L, _L)]
                    t = x * inv + bias
                    t = jnp.minimum(jnp.maximum(t, zero), ubound)
                    idx_f = (t + magic) - magic
                    buf[r, pl.ds(i * _L, _L)] = c0 + idx_f * step
            hout.append(pltpu.async_copy(
                buf, out_hbm.at[pl.ds(row0 + j * rch, rch)], osems[j]))
        for h in hout:
            h.wait()

    return _quantize


def kernel(input, codebook):
    nrow, ncol = input.shape
    k = codebook.shape[0]
    return _make_quantize(nrow, ncol, k)(input, codebook)
